# baseline (speedup 1.0000x reference)
import jax
import jax.numpy as jnp
from jax.experimental import pallas as pl
from jax.experimental.pallas import tpu as pltpu

_K = 8


def _copy_body(x_ref, o_ref, *sems):
    rows = x_ref.shape[0]
    chunk = rows // _K
    copies = [
        pltpu.make_async_copy(
            x_ref.at[pl.ds(i * chunk, chunk)],
            o_ref.at[pl.ds(i * chunk, chunk)],
            sems[i],
        )
        for i in range(_K)
    ]
    for c in copies:
        c.start()
    for c in copies:
        c.wait()


def kernel(x):
    b, s, d = x.shape
    x2 = x.reshape(b * s, d)
    out = pl.pallas_call(
        _copy_body,
        in_specs=[pl.BlockSpec(memory_space=pl.ANY)],
        out_specs=pl.BlockSpec(memory_space=pl.ANY),
        out_shape=jax.ShapeDtypeStruct((b * s, d), x.dtype),
        scratch_shapes=[pltpu.SemaphoreType.DMA] * _K,
    )(x2)
    return out.reshape(b, s, d)


# speedup vs baseline: 44.0555x; 44.0555x over previous
import jax
import jax.numpy as jnp
from jax.experimental import pallas as pl


def _copy_body(x_ref, o_ref):
    o_ref[...] = x_ref[...]


def kernel(x):
    b, s, d = x.shape
    x2 = x.reshape(b * s, d)
    rows = b * s
    block_rows = 256
    grid = (rows // block_rows,)
    out = pl.pallas_call(
        _copy_body,
        grid=grid,
        in_specs=[pl.BlockSpec((block_rows, d), lambda i: (i, 0))],
        out_specs=pl.BlockSpec((block_rows, d), lambda i: (i, 0)),
        out_shape=jax.ShapeDtypeStruct((rows, d), x.dtype),
    )(x2)
    return out.reshape(b, s, d)


# speedup vs baseline: 49.0393x; 1.1131x over previous
import jax
import jax.numpy as jnp
from jax.experimental import pallas as pl


def _copy_body(x_ref, o_ref):
    o_ref[...] = x_ref[...]


def kernel(x):
    b, s, d = x.shape
    x2 = x.reshape(b * s, d)
    rows = b * s
    block_rows = 1024
    grid = (rows // block_rows,)
    out = pl.pallas_call(
        _copy_body,
        grid=grid,
        in_specs=[pl.BlockSpec((block_rows, d), lambda i: (i, 0))],
        out_specs=pl.BlockSpec((block_rows, d), lambda i: (i, 0)),
        out_shape=jax.ShapeDtypeStruct((rows, d), x.dtype),
    )(x2)
    return out.reshape(b, s, d)
